# baseline (speedup 1.0000x reference)
import functools

import jax
import jax.numpy as jnp
from jax import lax
from jax.experimental import pallas as pl
from jax.experimental.pallas import tpu as pltpu
from jax.experimental.pallas import tpu_sc as plsc

NUM_WORKERS = 32
SPLIT = 128
NBUF = 4


def _make_lookup(b, s, d):
  rows_per_w = b // NUM_WORKERS
  mesh = plsc.VectorSubcoreMesh(core_axis_name="c", subcore_axis_name="s")

  @functools.partial(
      pl.kernel,
      out_type=jax.ShapeDtypeStruct((b, s, d), jnp.float32),
      mesh=mesh,
      compiler_params=pltpu.CompilerParams(use_tc_tiling_on_sc=True),
      scratch_types=[
          pltpu.VMEM((rows_per_w, s), jnp.int32),
          pltpu.VMEM((NBUF, s, d), jnp.float32),
          pltpu.SemaphoreType.DMA,
          pltpu.SemaphoreType.DMA,
      ],
  )
  def lookup(idx_hbm, table_hbm, out_hbm, idx_v, rows_v, gsem, wsem):
    wid = lax.axis_index("s") * 2 + lax.axis_index("c")
    base = wid * rows_per_w
    pltpu.sync_copy(idx_hbm.at[pl.ds(base, rows_per_w)], idx_v)

    def gathers(r, slot):
      pltpu.async_copy(
          table_hbm.at[idx_v.at[r, pl.ds(0, SPLIT)]],
          rows_v.at[slot, pl.ds(0, SPLIT)],
          gsem,
      )
      pltpu.async_copy(
          table_hbm.at[idx_v.at[r, pl.ds(SPLIT, s - SPLIT)]],
          rows_v.at[slot, pl.ds(SPLIT, s - SPLIT)],
          gsem,
      )

    def wait_gathers(r, slot):
      pltpu.make_async_copy(
          table_hbm.at[idx_v.at[r, pl.ds(0, SPLIT)]],
          rows_v.at[slot, pl.ds(0, SPLIT)],
          gsem,
      ).wait()
      pltpu.make_async_copy(
          table_hbm.at[idx_v.at[r, pl.ds(SPLIT, s - SPLIT)]],
          rows_v.at[slot, pl.ds(SPLIT, s - SPLIT)],
          gsem,
      ).wait()

    for p in range(NBUF - 1):
      gathers(p, p)

    def body(r, carry):
      slot = lax.rem(r, NBUF)
      ahead = r + NBUF - 1
      aslot = lax.rem(ahead, NBUF)

      @pl.when(jnp.logical_and(r >= 1, ahead < rows_per_w))
      def _():
        pltpu.make_async_copy(
            rows_v.at[aslot], out_hbm.at[base + r - 1], wsem
        ).wait()

      @pl.when(ahead < rows_per_w)
      def _():
        gathers(ahead, aslot)

      wait_gathers(r, slot)
      pltpu.async_copy(rows_v.at[slot], out_hbm.at[base + r], wsem)
      return carry

    lax.fori_loop(0, rows_per_w, body, 0)
    for p in range(NBUF):
      pltpu.make_async_copy(rows_v.at[p], out_hbm.at[base], wsem).wait()

  return lookup


def kernel(x, table):
  b, s = x.shape
  v, d = table.shape
  return _make_lookup(b, s, d)(x, table)


# speedup vs baseline: 1.0052x; 1.0052x over previous
import functools

import jax
import jax.numpy as jnp
from jax import lax
from jax.experimental import pallas as pl
from jax.experimental.pallas import tpu as pltpu
from jax.experimental.pallas import tpu_sc as plsc

NUM_WORKERS = 32
SPLIT = 128
NBUF = 2


def _make_lookup(b, s, d):
  rows_per_w = b // NUM_WORKERS
  mesh = plsc.VectorSubcoreMesh(core_axis_name="c", subcore_axis_name="s")

  @functools.partial(
      pl.kernel,
      out_type=jax.ShapeDtypeStruct((b, s, d), jnp.float32),
      mesh=mesh,
      scratch_types=[
          pltpu.VMEM((rows_per_w, s), jnp.int32),
          pltpu.VMEM((NBUF, 2, s, d), jnp.float32),
          pltpu.SemaphoreType.DMA,
          pltpu.SemaphoreType.DMA,
      ],
  )
  def lookup(idx_hbm, table_hbm, out_hbm, idx_v, rows_v, gsem, wsem):
    wid = lax.axis_index("s") * 2 + lax.axis_index("c")
    base = wid * rows_per_w
    pltpu.sync_copy(idx_hbm.at[pl.ds(base, rows_per_w)], idx_v)

    def gathers(p, slot):
      for h in range(2):
        pltpu.async_copy(
            table_hbm.at[idx_v.at[2 * p + h, pl.ds(0, SPLIT)]],
            rows_v.at[slot, h, pl.ds(0, SPLIT)],
            gsem,
        )
        pltpu.async_copy(
            table_hbm.at[idx_v.at[2 * p + h, pl.ds(SPLIT, s - SPLIT)]],
            rows_v.at[slot, h, pl.ds(SPLIT, s - SPLIT)],
            gsem,
        )

    def wait_gathers(p, slot):
      for h in range(2):
        pltpu.make_async_copy(
            table_hbm.at[idx_v.at[2 * p + h, pl.ds(0, SPLIT)]],
            rows_v.at[slot, h, pl.ds(0, SPLIT)],
            gsem,
        ).wait()
        pltpu.make_async_copy(
            table_hbm.at[idx_v.at[2 * p + h, pl.ds(SPLIT, s - SPLIT)]],
            rows_v.at[slot, h, pl.ds(SPLIT, s - SPLIT)],
            gsem,
        ).wait()

    for p in range(NBUF - 1):
      gathers(p, p)

    n_pairs = rows_per_w // 2

    def body(p, carry):
      slot = lax.rem(p, NBUF)
      ahead = p + NBUF - 1
      aslot = lax.rem(ahead, NBUF)

      @pl.when(jnp.logical_and(p >= 1, ahead < n_pairs))
      def _():
        pltpu.make_async_copy(
            rows_v.at[aslot], out_hbm.at[pl.ds(base + 2 * (p - 1), 2)], wsem
        ).wait()

      @pl.when(ahead < n_pairs)
      def _():
        gathers(ahead, aslot)

      wait_gathers(p, slot)
      pltpu.async_copy(
          rows_v.at[slot], out_hbm.at[pl.ds(base + 2 * p, 2)], wsem
      )
      return carry

    lax.fori_loop(0, n_pairs, body, 0)
    for p in range(NBUF):
      pltpu.make_async_copy(
          rows_v.at[p], out_hbm.at[pl.ds(base, 2)], wsem
      ).wait()

  return lookup


def kernel(x, table):
  b, s = x.shape
  v, d = table.shape
  return _make_lookup(b, s, d)(x, table)
